# baseline (device time: 12917 ns/iter reference)
import jax
import jax.numpy as jnp
from jax import lax
from jax.experimental import pallas as pl
from jax.experimental.pallas import tpu as pltpu

N_DEV = 32
PLANE = 8
NZ = 4


def kernel(x):
    m_per, n = x.shape

    def body(x_ref, out_ref, comm_ref, zcomm_ref, send_sems, recv_sems):
        my = lax.axis_index("i")
        z = my // PLANE
        r = lax.rem(my, PLANE)

        barrier_sem = pltpu.get_barrier_semaphore()
        for o in range(1, PLANE):
            tgt = z * PLANE + lax.rem(r + o, PLANE)
            pl.semaphore_signal(
                barrier_sem, inc=1,
                device_id=(tgt,), device_id_type=pl.DeviceIdType.MESH,
            )
        for q in range(1, NZ):
            tgt = lax.rem(z + q, NZ) * PLANE + r
            pl.semaphore_signal(
                barrier_sem, inc=1,
                device_id=(tgt,), device_id_type=pl.DeviceIdType.MESH,
            )

        comm_ref[0, :] = jnp.max(x_ref[:, :], axis=0)

        pl.semaphore_wait(barrier_sem, PLANE + NZ - 2)

        a_rdmas = []
        for o in range(1, PLANE):
            tgt = z * PLANE + lax.rem(r + o, PLANE)
            rdma = pltpu.make_async_remote_copy(
                src_ref=comm_ref.at[0],
                dst_ref=comm_ref.at[o],
                send_sem=send_sems.at[o],
                recv_sem=recv_sems.at[o],
                device_id=(tgt,),
                device_id_type=pl.DeviceIdType.MESH,
            )
            rdma.start()
            a_rdmas.append(rdma)
        for rdma in a_rdmas:
            rdma.wait()

        zcomm_ref[0, :] = jnp.max(comm_ref[:, :], axis=0)

        b_rdmas = []
        for q in range(1, NZ):
            tgt = lax.rem(z + q, NZ) * PLANE + r
            rdma = pltpu.make_async_remote_copy(
                src_ref=zcomm_ref.at[0],
                dst_ref=zcomm_ref.at[q],
                send_sem=send_sems.at[PLANE - 1 + q],
                recv_sem=recv_sems.at[PLANE - 1 + q],
                device_id=(tgt,),
                device_id_type=pl.DeviceIdType.MESH,
            )
            rdma.start()
            b_rdmas.append(rdma)
        for rdma in b_rdmas:
            rdma.wait()

        out_ref[0, :] = jnp.max(zcomm_ref[:, :], axis=0)

    return pl.pallas_call(
        body,
        out_shape=jax.ShapeDtypeStruct((1, n), x.dtype),
        in_specs=[pl.BlockSpec(memory_space=pltpu.VMEM)],
        out_specs=pl.BlockSpec(memory_space=pltpu.VMEM),
        scratch_shapes=[
            pltpu.VMEM((PLANE, n), x.dtype),
            pltpu.VMEM((NZ, n), x.dtype),
            pltpu.SemaphoreType.DMA((PLANE + NZ - 1,)),
            pltpu.SemaphoreType.DMA((PLANE + NZ - 1,)),
        ],
        compiler_params=pltpu.CompilerParams(collective_id=0),
    )(x)


# device time: 11922 ns/iter; 1.0835x vs baseline; 1.0835x over previous
import jax
import jax.numpy as jnp
from jax import lax
from jax.experimental import pallas as pl
from jax.experimental.pallas import tpu as pltpu

N_DEV = 32


def kernel(x):
    m_per, n = x.shape

    def body(x_ref, out_ref, comm_ref, send_sems, recv_sems):
        my = lax.axis_index("i")

        barrier_sem = pltpu.get_barrier_semaphore()
        for o in range(1, N_DEV):
            tgt = lax.rem(my + o, N_DEV)
            pl.semaphore_signal(
                barrier_sem,
                inc=1,
                device_id=(tgt,),
                device_id_type=pl.DeviceIdType.MESH,
            )

        comm_ref[0, :] = jnp.max(x_ref[:, :], axis=0)

        pl.semaphore_wait(barrier_sem, N_DEV - 1)

        rdmas = []
        for o in range(1, N_DEV):
            tgt = lax.rem(my + o, N_DEV)
            rdma = pltpu.make_async_remote_copy(
                src_ref=comm_ref.at[0],
                dst_ref=comm_ref.at[o],
                send_sem=send_sems.at[o],
                recv_sem=recv_sems.at[o],
                device_id=(tgt,),
                device_id_type=pl.DeviceIdType.MESH,
            )
            rdma.start()
            rdmas.append(rdma)

        for rdma in rdmas:
            rdma.wait()

        out_ref[0, :] = jnp.max(comm_ref[:, :], axis=0)

    return pl.pallas_call(
        body,
        out_shape=jax.ShapeDtypeStruct((1, n), x.dtype),
        in_specs=[pl.BlockSpec(memory_space=pltpu.VMEM)],
        out_specs=pl.BlockSpec(memory_space=pltpu.VMEM),
        scratch_shapes=[
            pltpu.VMEM((N_DEV, n), x.dtype),
            pltpu.SemaphoreType.DMA((N_DEV,)),
            pltpu.SemaphoreType.DMA((N_DEV,)),
        ],
        compiler_params=pltpu.CompilerParams(collective_id=0),
    )(x)
